# baseline (speedup 1.0000x reference)
import jax
import jax.numpy as jnp
from jax import lax
from jax.experimental import pallas as pl
from jax.experimental.pallas import tpu as pltpu
from jax.experimental.pallas import tpu_sc as plsc

VOCAB = 1000000
D = 64
BATCH = 4096
SEQ = 200

NC = 2
NS = 16
NW = NC * NS
B_PER_W = BATCH // NW
NB = 4
AHEAD = 2


def _gather_body(x_hbm, table_hbm, out_hbm, idx_v, r0, r1, r2, r3,
                 g0, g1, g2, g3, s0, s1, s2, s3):
    rows = [r0, r1, r2, r3]
    gs = [g0, g1, g2, g3]
    ss = [s0, s1, s2, s3]
    wid = lax.axis_index("s") * NC + lax.axis_index("c")
    b_base = wid * B_PER_W

    pltpu.sync_copy(x_hbm.at[pl.ds(b_base, B_PER_W)], idx_v)

    def gather(j, buf, sem):
        pltpu.async_copy(table_hbm.at[idx_v.at[j]], buf, sem)

    def out_slice(j):
        return out_hbm.at[b_base + j]

    for j in range(AHEAD):
        gather(j, rows[j % NB], gs[j % NB])

    @pl.loop(0, B_PER_W, step=NB)
    def _block(i):
        for bb in range(NB):
            j = i + bb
            ga = j + AHEAD
            gb = (bb + AHEAD) % NB

            @pl.when(ga < B_PER_W)
            def _issue():
                @pl.when(ga >= NB)
                def _wait_sc():
                    pltpu.make_async_copy(rows[gb], out_slice(ga - NB),
                                          ss[gb]).wait()
                gather(ga, rows[gb], gs[gb])

            pltpu.make_async_copy(table_hbm.at[idx_v.at[j]], rows[bb],
                                  gs[bb]).wait()
            pltpu.async_copy(rows[bb], out_slice(j), ss[bb])

    for bb in range(NB):
        j = B_PER_W - NB + bb
        pltpu.make_async_copy(rows[j % NB], out_slice(j), ss[j % NB]).wait()


@jax.jit
def _sc_gather(x, table):
    mesh = plsc.VectorSubcoreMesh(core_axis_name="c", subcore_axis_name="s")
    return pl.kernel(
        _gather_body,
        out_type=jax.ShapeDtypeStruct((BATCH, SEQ, D), jnp.float32),
        mesh=mesh,
        scratch_types=(
            [pltpu.VMEM((B_PER_W, SEQ), jnp.int32)]
            + [pltpu.VMEM((SEQ, D), jnp.float32) for _ in range(NB)]
            + [pltpu.SemaphoreType.DMA for _ in range(2 * NB)]
        ),
        compiler_params=pltpu.CompilerParams(use_tc_tiling_on_sc=False),
    )(x, table)


def kernel(x, table):
    return _sc_gather(x, table)


# speedup vs baseline: 1.2229x; 1.2229x over previous
import jax
import jax.numpy as jnp
from jax import lax
from jax.experimental import pallas as pl
from jax.experimental.pallas import tpu as pltpu
from jax.experimental.pallas import tpu_sc as plsc

VOCAB = 1000000
D = 64
DP = 128
BATCH = 4096
SEQ = 200
N = BATCH * SEQ

NC = 2
NS = 16
NW = NC * NS
B_PER_W = BATCH // NW
PER_W = B_PER_W * SEQ
NB = 4
AHEAD = 2


def _gather_body(xf_hbm, tpad_hbm, out_hbm, idx_v, r0, r1, r2, r3,
                 g0, g1, g2, g3, s0, s1, s2, s3):
    rows = [r0, r1, r2, r3]
    gs = [g0, g1, g2, g3]
    ss = [s0, s1, s2, s3]
    wid = lax.axis_index("s") * NC + lax.axis_index("c")
    b_base = wid * B_PER_W

    pltpu.sync_copy(xf_hbm.at[pl.ds(wid * PER_W, PER_W)], idx_v)

    def gather(j, buf, sem):
        pltpu.async_copy(tpad_hbm.at[idx_v.at[pl.ds(j * SEQ, SEQ)]], buf, sem)

    def out_slice(j):
        return out_hbm.at[b_base + j]

    for j in range(AHEAD):
        gather(j, rows[j % NB], gs[j % NB])

    @pl.loop(0, B_PER_W, step=NB)
    def _block(i):
        for bb in range(NB):
            j = i + bb
            ga = j + AHEAD
            gb = (bb + AHEAD) % NB

            @pl.when(ga < B_PER_W)
            def _issue():
                @pl.when(ga >= NB)
                def _wait_sc():
                    pltpu.make_async_copy(rows[gb], out_slice(ga - NB),
                                          ss[gb]).wait()
                gather(ga, rows[gb], gs[gb])

            pltpu.make_async_copy(tpad_hbm.at[idx_v.at[pl.ds(j * SEQ, SEQ)]],
                                  rows[bb], gs[bb]).wait()
            pltpu.async_copy(rows[bb], out_slice(j), ss[bb])

    for bb in range(NB):
        j = B_PER_W - NB + bb
        pltpu.make_async_copy(rows[j % NB], out_slice(j), ss[j % NB]).wait()


@jax.jit
def _sc_gather(xf, tpad):
    mesh = plsc.VectorSubcoreMesh(core_axis_name="c", subcore_axis_name="s")
    return pl.kernel(
        _gather_body,
        out_type=jax.ShapeDtypeStruct((BATCH, SEQ, DP), jnp.float32),
        mesh=mesh,
        scratch_types=(
            [pltpu.VMEM((PER_W,), jnp.int32)]
            + [pltpu.VMEM((SEQ, DP), jnp.float32) for _ in range(NB)]
            + [pltpu.SemaphoreType.DMA for _ in range(2 * NB)]
        ),
        compiler_params=pltpu.CompilerParams(use_tc_tiling_on_sc=True),
    )(xf, tpad)


def kernel(x, table):
    tpad = jnp.pad(table, ((0, 0), (0, DP - D)))
    out = _sc_gather(x.reshape(N), tpad)
    return out[:, :, :D]
